# baseline (speedup 1.0000x reference)
import functools

import jax
import jax.numpy as jnp
from jax import lax
from jax.experimental import pallas as pl
from jax.experimental.pallas import tpu as pltpu
from jax.experimental.pallas import tpu_sc as plsc

NC = 2
NS = 16
NW = NC * NS
LANES = 16
CH = 12800
SLOT = 13312


def _prep_kernel(x, ei, n_pad, nch):
    def body(x_ref, ei_ref, rs_ref, f_out, t_out):
        rs_ref[...] = jnp.sum(x_ref[...], axis=-1)
        f_out[pl.ds(0, CH)] = ei_ref[0]
        t_out[pl.ds(0, CH)] = ei_ref[1]

    d = x.shape[1]
    blk_n = n_pad // nch
    n_blocks_valid = -(-x.shape[0] // blk_n)
    return pl.pallas_call(
        body,
        grid=(nch,),
        in_specs=[
            pl.BlockSpec((blk_n, d),
                         lambda i: (jnp.minimum(i, n_blocks_valid - 1), 0)),
            pl.BlockSpec((2, CH), lambda i: (0, i)),
        ],
        out_specs=[
            pl.BlockSpec((blk_n,), lambda i: (i,)),
            pl.BlockSpec((SLOT,), lambda i: (i,)),
            pl.BlockSpec((SLOT,), lambda i: (i,)),
        ],
        out_shape=[
            jax.ShapeDtypeStruct((n_pad,), jnp.float32),
            jax.ShapeDtypeStruct((nch * SLOT,), jnp.int32),
            jax.ShapeDtypeStruct((nch * SLOT,), jnp.int32),
        ],
    )(x, ei)


def _fused_sc_kernel(f1d, t1d, rs_pad, *, e, n_pad, nch, seg):
    mesh = plsc.VectorSubcoreMesh(core_axis_name="c", subcore_axis_name="s")

    @functools.partial(
        pl.kernel,
        mesh=mesh,
        out_type=jax.ShapeDtypeStruct((e,), jnp.float32),
        scratch_types=[
            pltpu.VMEM((CH,), jnp.int32),
            pltpu.VMEM((CH,), jnp.float32),
            pltpu.VMEM((seg,), jnp.float32),
            pltpu.VMEM((seg,), jnp.float32),
            pltpu.VMEM((n_pad,), jnp.float32),
            pltpu.VMEM((n_pad,), jnp.float32),
            pltpu.VMEM((CH,), jnp.int32),
            pltpu.VMEM((CH,), jnp.int32),
            pltpu.VMEM((CH,), jnp.float32),
            pltpu.VMEM_SHARED((n_pad,), jnp.float32),
            pltpu.VMEM_SHARED((n_pad,), jnp.float32),
            pltpu.VMEM_SHARED((n_pad,), jnp.float32),
            pltpu.SemaphoreType.DMA,
            pltpu.SemaphoreType.DMA,
            pltpu.SemaphoreType.DMA,
            pltpu.SemaphoreType.DMA,
        ],
        compiler_params=pltpu.CompilerParams(needs_layout_passes=False),
    )
    def k(f_hbm, t_hbm, rs_hbm, out_hbm, idx_v, ones_v, seg_v, rs_v, s_v,
          tt_v, fi_v, ti_v, o_v, deg_sh, s_sh, t_sh, sem_h, sem_f, sem_t,
          sem_r):
        cid = lax.axis_index("c")
        sid = lax.axis_index("s")
        wid = cid * NS + sid
        sbase = pl.multiple_of(sid * seg, 8)
        gbase = pl.multiple_of(wid * SLOT, 8)
        hbase = pl.multiple_of(sid * SLOT, 8)

        hist_cp = pltpu.async_copy(t_hbm.at[pl.ds(hbase, CH)], idx_v, sem_h)
        rs_cp = pltpu.async_copy(rs_hbm.at[pl.ds(sbase, seg)], rs_v, sem_r)

        @pl.when(wid < nch)
        def _():
            pltpu.async_copy(f_hbm.at[pl.ds(gbase, CH)], fi_v, sem_f)
            pltpu.async_copy(t_hbm.at[pl.ds(gbase, CH)], ti_v, sem_t)

        ones16 = jnp.full((LANES,), 1.0, jnp.float32)
        zeros16 = jnp.zeros((LANES,), jnp.float32)

        @plsc.parallel_loop(0, seg // LANES, unroll=8)
        def _(i):
            seg_v[pl.ds(i * LANES, LANES)] = zeros16

        pltpu.sync_copy(seg_v, deg_sh.at[pl.ds(sbase, seg)])

        @plsc.parallel_loop(0, CH // LANES, unroll=8)
        def _(i):
            ones_v[pl.ds(i * LANES, LANES)] = ones16

        hist_cp.wait()
        plsc.subcore_barrier()

        pltpu.sync_copy(ones_v, deg_sh.at[idx_v], add=True)

        @pl.when(sid + NS < nch)
        def _():
            h2 = pl.multiple_of((sid + NS) * SLOT, 8)
            pltpu.sync_copy(t_hbm.at[pl.ds(h2, CH)], idx_v)
            pltpu.sync_copy(ones_v, deg_sh.at[idx_v], add=True)

        plsc.subcore_barrier()

        pltpu.sync_copy(deg_sh.at[pl.ds(sbase, seg)], seg_v)
        rs_cp.wait()

        @plsc.parallel_loop(0, seg // LANES, unroll=4)
        def _(i):
            sl = pl.ds(i * LANES, LANES)
            d = seg_v[sl]
            ibits = plsc.bitcast(d, jnp.int32)
            y = plsc.bitcast(
                jnp.full((LANES,), 0x5F3759DF, jnp.int32)
                - lax.shift_right_logical(ibits, 1),
                jnp.float32,
            )
            hd = 0.5 * d
            y = y * (1.5 - hd * y * y)
            y = y * (1.5 - hd * y * y)
            y = y * (1.5 - hd * y * y)
            t = jnp.where(d == 0.0, 0.0, y)
            seg_v[sl] = t
            rs_v[sl] = t * rs_v[sl]

        pltpu.sync_copy(rs_v, s_sh.at[pl.ds(sbase, seg)])
        pltpu.sync_copy(seg_v, t_sh.at[pl.ds(sbase, seg)])
        plsc.subcore_barrier()

        @pl.when(wid < nch)
        def _():
            pltpu.sync_copy(s_sh, s_v)
            pltpu.sync_copy(t_sh, tt_v)
            pltpu.make_async_copy(f_hbm.at[pl.ds(gbase, CH)], fi_v,
                                  sem_f).wait()
            pltpu.make_async_copy(t_hbm.at[pl.ds(gbase, CH)], ti_v,
                                  sem_t).wait()

            @plsc.parallel_loop(0, CH // LANES, unroll=8)
            def _(i):
                sl = pl.ds(i * LANES, LANES)
                sv = plsc.load_gather(s_v, [fi_v[sl]])
                tv = plsc.load_gather(tt_v, [ti_v[sl]])
                o_v[sl] = sv * tv

            obase = pl.multiple_of(wid * CH, 8)
            pltpu.sync_copy(o_v, out_hbm.at[pl.ds(obase, CH)])

    return k(f1d, t1d, rs_pad)


def kernel(x, edge_index):
    n, d = x.shape
    e = edge_index.shape[1]
    assert e % CH == 0, "edge count must split into chunks"
    nch = e // CH
    assert nch <= NW and NS <= nch
    n_pad = -(-n // (NS * LANES)) * (NS * LANES)
    n_pad = -(-n_pad // nch) * nch
    while n_pad % (NS * LANES) != 0:
        n_pad += nch
    seg = n_pad // NS

    rs_pad, f1d, t1d = _prep_kernel(x, edge_index.astype(jnp.int32),
                                    n_pad, nch)
    return _fused_sc_kernel(f1d, t1d, rs_pad, e=e, n_pad=n_pad, nch=nch,
                            seg=seg)
